# baseline (speedup 1.0000x reference)
import functools

import jax
import jax.numpy as jnp
from jax import lax
from jax.experimental import pallas as pl
from jax.experimental.pallas import tpu as pltpu
from jax.experimental.pallas import tpu_sc as plsc

B, N, H, W, C = 16, 32, 512, 512, 3
PH, PW = 64, 64
ASPECT = 1.0
ORIGIN = (0.5, 0.5)
SCALE = 0.2

MIN_S = int(200.0 * SCALE)
MAX_S = int(380.0 * SCALE)
SIZES = list(range(MIN_S, MAX_S + 1))
NSIZES = len(SIZES)
ROWPAD = 240
PREFIX = [0]
for _s in SIZES:
    PREFIX.append(PREFIX[-1] + _s)
TOTROWS = PREFIX[-1]
R_ROWS = TOTROWS + 80

WC = W * C
CHUNK = 32
NCHUNKS = (H // 2) // CHUNK


def _resize_mats():
    mats = []
    eye = jnp.eye(PH, dtype=jnp.float32)
    for s in SIZES:
        m = jax.image.resize(eye, (s, PH), method="bilinear")
        mats.append(jnp.pad(m, ((0, MAX_S - s), (0, 0))))
    w = jnp.stack(mats)
    wt = jnp.transpose(w, (0, 2, 1))
    k4 = jnp.zeros((NSIZES, PH, C, ROWPAD // C, C), jnp.float32)
    for c in range(C):
        k4 = k4.at[:, :, c, :MAX_S, c].set(wt)
    k = k4.reshape(NSIZES, C * PH, ROWPAD)
    return w.reshape(NSIZES * MAX_S, PH), k


def _prep_body(patch_ref, w_ref, k_ref, boxes_ref, r_ref, pbf_ref, meta_ref):
    f32 = jnp.float32
    r_ref[...] = jnp.zeros(r_ref.shape, f32)

    p2 = patch_ref[...].reshape(PH, C * PW)
    hi = jax.lax.Precision.HIGHEST
    t = jax.lax.dot(w_ref[...], p2, precision=hi)
    for si in range(NSIZES):
        rows = jax.lax.dot(t[si * MAX_S:(si + 1) * MAX_S], k_ref[si],
                           precision=hi)
        r_ref[pl.ds(PREFIX[si], MAX_S), :] = rows

    bb = boxes_ref[...]
    ymin, xmin = bb[..., 0], bb[..., 1]
    h = bb[..., 2] - ymin
    w = bb[..., 3] - xmin
    patch_h = h * SCALE
    patch_w = ASPECT * patch_h
    ymin_p = ymin + h * ORIGIN[1]
    xmin_p = xmin + w * ORIGIN[0]
    ymin_p = jnp.where(ymin_p + patch_h > float(H), float(H) - patch_h, ymin_p)
    xmin_p = jnp.where(xmin_p + patch_w > float(W), float(W) - patch_w, xmin_p)
    pbf = jnp.stack([ymin_p, xmin_p, patch_h, patch_w], axis=-1)
    pbf_ref[...] = pbf

    pb = pbf.astype(jnp.int32)
    ph, pw = pb[..., 2], pb[..., 3]
    keep = ((ph * pw) > 900).astype(jnp.int32)
    k = jnp.clip(jnp.minimum(ph, pw) - MIN_S, 0, NSIZES - 1)
    s_used = MIN_S + k
    y = jnp.clip(pb[..., 0], 0, H - s_used)
    x = jnp.clip(pb[..., 1], 0, W - s_used)
    rowbase = MIN_S * k + (k * (k - 1)) // 2
    zeros = jnp.zeros_like(y)
    meta_ref[...] = jnp.stack(
        [y, 3 * x, 3 * s_used, rowbase, keep, s_used, zeros, zeros], axis=1)


@jax.jit
def _prep(patch, w, k, boxes):
    return pl.pallas_call(
        _prep_body,
        out_shape=[
            jax.ShapeDtypeStruct((R_ROWS, ROWPAD), jnp.float32),
            jax.ShapeDtypeStruct((B, N, 4), jnp.float32),
            jax.ShapeDtypeStruct((B, 8, N), jnp.int32),
        ],
    )(patch, w, k, boxes)


def _sc_body(images_hbm, rstack_hbm, meta_hbm, out_hbm, buf, pbuf, mbuf_v):
    i32 = jnp.int32
    cid = lax.axis_index("c")
    sid = lax.axis_index("s")
    b = sid
    half = cid
    iota = lax.iota(i32, 16)

    pltpu.sync_copy(meta_hbm.at[pl.ds(b * 256, 256)],
                    mbuf_v.at[pl.ds(0, 8 * N)])

    def get(f, n):
        return mbuf_v[pl.ds(f * 32 + n, 16)][0]

    def chunk_body(cI, carry):
        rowstart = half * (H // 2) + cI * CHUNK
        img_off = (b * H + rowstart) * WC
        pltpu.sync_copy(images_hbm.at[pl.ds(img_off, CHUNK * WC)], buf)

        def box_body(n, carry2):
            y = get(0, n)
            x3 = get(1, n)
            s3 = get(2, n)
            rowbase = get(3, n)
            keep = get(4, n)
            s = get(5, n)
            ov_lo = jnp.maximum(y, rowstart)
            ov_hi = jnp.minimum(y + s, rowstart + CHUNK)

            @pl.when((ov_hi > ov_lo) & (keep > 0))
            def _():
                r0 = ov_lo - y
                pltpu.sync_copy(
                    rstack_hbm.at[pl.ds((rowbase + r0) * ROWPAD,
                                        CHUNK * ROWPAD)], pbuf)

                def row_body(r, carry3):
                    lr = ov_lo - rowstart + r
                    dbase = lr * WC + x3
                    nk = (s3 + 15) // 16

                    def k_body(kk, carry4):
                        off = kk * 16
                        vals = pbuf[pl.ds(r * ROWPAD + off, 16)]
                        idx = dbase + off + iota
                        mask = (off + iota) < s3
                        plsc.store_scatter(buf, [idx], vals, mask=mask)
                        return carry4

                    return lax.fori_loop(0, nk, k_body, carry3)

                lax.fori_loop(0, ov_hi - ov_lo, row_body, 0)

            return carry2

        lax.fori_loop(0, N, box_body, 0)
        pltpu.sync_copy(buf, out_hbm.at[pl.ds(img_off, CHUNK * WC)])
        return carry

    lax.fori_loop(0, NCHUNKS, chunk_body, 0)


@jax.jit
def _scatter(images_flat, rstack_flat, meta_flat):
    mesh = plsc.VectorSubcoreMesh(core_axis_name="c", subcore_axis_name="s")
    fn = functools.partial(
        pl.kernel,
        out_type=jax.ShapeDtypeStruct((B * H * WC,), jnp.float32),
        mesh=mesh,
        compiler_params=pltpu.CompilerParams(needs_layout_passes=False),
        scratch_types=[
            pltpu.VMEM((CHUNK * WC,), jnp.float32),
            pltpu.VMEM((CHUNK * ROWPAD,), jnp.float32),
            pltpu.VMEM((8 * N + 16,), jnp.int32),
        ],
    )(_sc_body)
    return fn(images_flat, rstack_flat, meta_flat)


def kernel(batch_boxes, images, patch):
    w, k = _resize_mats()
    rstack, patch_boxes, meta = _prep(patch, w, k, batch_boxes)
    out_flat = _scatter(images.reshape(-1), rstack.reshape(-1),
                        meta.reshape(-1))
    imgs = out_flat.reshape(B, H, W, C)
    td = jax.random.randint(jax.random.key(123), (B, N, 3), 0, 2).astype(bool)
    return patch_boxes, td, imgs


# speedup vs baseline: 23.8145x; 23.8145x over previous
import functools

import jax
import jax.numpy as jnp
import numpy as np
from jax import lax
from jax.experimental import pallas as pl
from jax.experimental.pallas import tpu as pltpu
from jax.experimental.pallas import tpu_sc as plsc

B, N, H, W, C = 16, 32, 512, 512, 3
PH, PW = 64, 64
ASPECT = 1.0
ORIGIN = (0.5, 0.5)
SCALE = 0.2

MIN_S = int(200.0 * SCALE)
MAX_S = int(380.0 * SCALE)
SIZES = list(range(MIN_S, MAX_S + 1))
NSIZES = len(SIZES)

R3_ROWS = NSIZES * C * MAX_S + 44

CHUNK = 32
NCHUNKS = (H // 2) // CHUNK
PLANE = H * W
TRW = 4 * 8 * 128
CHUNKF = (CHUNK // 8) * TRW


def _weight_mat_np(in_size, out_size):
    f32 = np.float32
    inv = f32(in_size / out_size)
    kscale = f32(max(float(inv), 1.0))
    sample_f = (np.arange(out_size, dtype=f32) + f32(0.5)) * inv - f32(0.5)
    x = np.abs(sample_f[None, :]
               - np.arange(in_size, dtype=f32)[:, None]) / kscale
    w = np.maximum(f32(0), f32(1) - x.astype(f32)).astype(f32)
    tot = w.sum(0, keepdims=True, dtype=f32)
    w = np.where(np.abs(tot) > 1000.0 * np.finfo(np.float32).eps,
                 w / np.where(tot != 0, tot, 1), 0).astype(f32)
    valid = (sample_f >= -0.5) & (sample_f <= in_size - 0.5)
    w = np.where(valid[None, :], w, 0).astype(f32)
    return w.T


def _resize_mats():
    mats = []
    for s in SIZES:
        m = _weight_mat_np(PH, s)
        mats.append(np.pad(m, ((0, MAX_S - s), (0, 0))))
    w = np.stack(mats).astype(np.float32)
    wt = np.transpose(w, (0, 2, 1))
    k5 = np.zeros((NSIZES, C, PH, C, 128), np.float32)
    for c in range(C):
        k5[:, c, :, c, :MAX_S] = wt
    kp = k5.reshape(NSIZES, C * PH, C * 128)
    return w.reshape(NSIZES * MAX_S, PH), kp


_W_NP, _KP_NP = _resize_mats()


def _prep_body(patch_ref, w_ref, kp_ref, boxes_ref, r_ref, pbf_ref, meta_ref):
    f32 = jnp.float32
    r_ref[...] = jnp.zeros(r_ref.shape, f32)

    p = patch_ref[...]
    pcat = jnp.concatenate([p[:, :, c] for c in range(C)], axis=1)
    hi = jax.lax.Precision.HIGHEST
    t = jax.lax.dot(w_ref[...], pcat, precision=hi)
    for si in range(NSIZES):
        o = jax.lax.dot(t[si * MAX_S:(si + 1) * MAX_S], kp_ref[si],
                        precision=hi)
        for c in range(C):
            r_ref[pl.ds((si * C + c) * MAX_S, MAX_S), :] = \
                o[:, 128 * c:128 * (c + 1)]

    bb = boxes_ref[...]
    ymin, xmin = bb[..., 0], bb[..., 1]
    h = bb[..., 2] - ymin
    w2 = bb[..., 3] - xmin
    patch_h = h * SCALE
    patch_w = ASPECT * patch_h
    ymin_p = ymin + h * ORIGIN[1]
    xmin_p = xmin + w2 * ORIGIN[0]
    ymin_p = jnp.where(ymin_p + patch_h > float(H), float(H) - patch_h, ymin_p)
    xmin_p = jnp.where(xmin_p + patch_w > float(W), float(W) - patch_w, xmin_p)
    pbf = jnp.stack([ymin_p, xmin_p, patch_h, patch_w], axis=-1)
    pbf_ref[...] = pbf

    pb = pbf.astype(jnp.int32)
    ph, pw = pb[..., 2], pb[..., 3]
    keep = ((ph * pw) > 900).astype(jnp.int32)
    k = jnp.clip(jnp.minimum(ph, pw) - MIN_S, 0, NSIZES - 1)
    s_used = MIN_S + k
    y = jnp.clip(pb[..., 0], 0, H - s_used)
    x = jnp.clip(pb[..., 1], 0, W - s_used)
    zeros = jnp.zeros_like(y)
    top = jnp.concatenate([y, x, s_used, k], axis=1)
    bot = jnp.concatenate([keep, zeros, zeros, zeros], axis=1)
    meta_ref[...] = jnp.concatenate([top, bot], axis=0)


@jax.jit
def _prep(patch, w, kp, boxes):
    return pl.pallas_call(
        _prep_body,
        out_shape=[
            jax.ShapeDtypeStruct((R3_ROWS, 128), jnp.float32),
            jax.ShapeDtypeStruct((B, N, 4), jnp.float32),
            jax.ShapeDtypeStruct((32, 128), jnp.int32),
        ],
    )(patch, w, kp, boxes)


def _sc_body(images_hbm, rstack_hbm, meta_hbm, out_hbm,
             buf0, buf1, buf2, pbuf, mbuf_v):
    i32 = jnp.int32
    cid = lax.axis_index("c")
    sid = lax.axis_index("s")
    b = sid
    half = cid
    iota = lax.iota(i32, 16)
    bufs = [buf0, buf1, buf2]

    pltpu.sync_copy(meta_hbm.at[pl.ds(b * 128, 128)],
                    mbuf_v.at[pl.ds(0, 128)])
    pltpu.sync_copy(meta_hbm.at[pl.ds((16 + b) * 128, 128)],
                    mbuf_v.at[pl.ds(128, 128)])

    def get(f, n):
        return mbuf_v[pl.ds(f * 32 + n, 16)][0]

    def chunk_body(cI, carry):
        rowstart = half * (H // 2) + cI * CHUNK
        troff = (rowstart // 8) * TRW
        for cch in range(C):
            pltpu.sync_copy(
                images_hbm.at[pl.ds((b * C + cch) * PLANE + troff, CHUNKF)],
                bufs[cch])

        def box_body(n, carry2):
            y = get(0, n)
            x = get(1, n)
            s = get(2, n)
            k = get(3, n)
            keep = get(4, n)
            ov_lo = jnp.maximum(y, rowstart)
            ov_hi = jnp.minimum(y + s, rowstart + CHUNK)

            @pl.when((ov_hi > ov_lo) & (keep > 0))
            def _():
                r0 = ov_lo - y
                for cch in range(C):
                    pltpu.sync_copy(
                        rstack_hbm.at[pl.ds(
                            ((k * C + cch) * MAX_S + r0) * 128, CHUNK * 128)],
                        pbuf.at[pl.ds(cch * CHUNK * 128, CHUNK * 128)])
                nk = (s + 15) // 16

                def row_body(r, carry3):
                    lr = ov_lo - rowstart + r
                    rowc = (lr // 8) * TRW + (lr % 8) * 128

                    def k_body(kk, carry4):
                        off = kk * 16
                        xx = x + off + iota
                        idx = rowc + (xx // 128) * 1024 + (xx % 128)
                        mask = (off + iota) < s
                        for cch in range(C):
                            vals = pbuf[pl.ds(
                                (cch * CHUNK + r) * 128 + off, 16)]
                            plsc.store_scatter(bufs[cch], [idx], vals,
                                               mask=mask)
                        return carry4

                    return lax.fori_loop(0, nk, k_body, carry3)

                lax.fori_loop(0, ov_hi - ov_lo, row_body, 0)

            return carry2

        lax.fori_loop(0, N, box_body, 0)
        for cch in range(C):
            pltpu.sync_copy(
                bufs[cch],
                out_hbm.at[pl.ds((b * C + cch) * PLANE + troff, CHUNKF)])
        return carry

    lax.fori_loop(0, NCHUNKS, chunk_body, 0)


@jax.jit
def _scatter(images_flat, rstack_flat, meta_flat):
    mesh = plsc.VectorSubcoreMesh(core_axis_name="c", subcore_axis_name="s")
    fn = functools.partial(
        pl.kernel,
        out_type=jax.ShapeDtypeStruct((B * C * PLANE,), jnp.float32),
        mesh=mesh,
        compiler_params=pltpu.CompilerParams(needs_layout_passes=False),
        scratch_types=[
            pltpu.VMEM((CHUNKF,), jnp.float32),
            pltpu.VMEM((CHUNKF,), jnp.float32),
            pltpu.VMEM((CHUNKF,), jnp.float32),
            pltpu.VMEM((C * CHUNK * 128,), jnp.float32),
            pltpu.VMEM((8 * N + 16,), jnp.int32),
        ],
    )(_sc_body)
    return fn(images_flat, rstack_flat, meta_flat)


def kernel(batch_boxes, images, patch):
    w = jnp.asarray(_W_NP)
    kp = jnp.asarray(_KP_NP)
    rstack, patch_boxes, meta = _prep(patch, w, kp, batch_boxes)

    img6 = (images.transpose(0, 3, 1, 2)
            .reshape(B, C, H // 8, 8, W // 128, 128)
            .transpose(0, 1, 2, 4, 3, 5))
    out_flat = _scatter(img6.reshape(-1), rstack.reshape(-1),
                        meta.reshape(-1))
    imgs = (out_flat.reshape(B, C, H // 8, W // 128, 8, 128)
            .transpose(0, 1, 2, 4, 3, 5)
            .reshape(B, C, H, W)
            .transpose(0, 2, 3, 1))
    td = jax.random.randint(jax.random.key(123), (B, N, 3), 0, 2).astype(bool)
    return patch_boxes, td, imgs


# speedup vs baseline: 31.6044x; 1.3271x over previous
import functools

import jax
import jax.numpy as jnp
import numpy as np
from jax import lax
from jax.experimental import pallas as pl
from jax.experimental.pallas import tpu as pltpu
from jax.experimental.pallas import tpu_sc as plsc

B, N, H, W, C = 16, 32, 512, 512, 3
PH, PW = 64, 64
ASPECT = 1.0
ORIGIN = (0.5, 0.5)
SCALE = 0.2

MIN_S = int(200.0 * SCALE)
MAX_S = int(380.0 * SCALE)
SIZES = list(range(MIN_S, MAX_S + 1))
NSIZES = len(SIZES)

RPAD = 80
PSTAGE = 40

CHUNK = 32
NCHUNKS = (H // 2) // CHUNK
PLANE = H * W
TRW = 4 * 8 * 128
CHUNKF = (CHUNK // 8) * TRW


def _weight_mat_np(in_size, out_size):
    f32 = np.float32
    inv = f32(in_size / out_size)
    kscale = f32(max(float(inv), 1.0))
    sample_f = (np.arange(out_size, dtype=f32) + f32(0.5)) * inv - f32(0.5)
    x = np.abs(sample_f[None, :]
               - np.arange(in_size, dtype=f32)[:, None]) / kscale
    w = np.maximum(f32(0), f32(1) - x.astype(f32)).astype(f32)
    tot = w.sum(0, keepdims=True, dtype=f32)
    w = np.where(np.abs(tot) > 1000.0 * np.finfo(np.float32).eps,
                 w / np.where(tot != 0, tot, 1), 0).astype(f32)
    valid = (sample_f >= -0.5) & (sample_f <= in_size - 0.5)
    w = np.where(valid[None, :], w, 0).astype(f32)
    return w.T


def _resize_mats():
    mats = []
    for s in SIZES:
        m = _weight_mat_np(PH, s)
        mats.append(np.pad(m, ((0, MAX_S - s), (0, 0))))
    w = np.stack(mats).astype(np.float32)
    wt = np.transpose(w, (0, 2, 1))
    k5 = np.zeros((NSIZES, C, PH, C, 128), np.float32)
    for c in range(C):
        k5[:, c, :, c, :MAX_S] = wt
    kp = k5.reshape(NSIZES, C * PH, C * 128)
    return w.reshape(NSIZES * MAX_S, PH), kp


_W_NP, _KP_NP = _resize_mats()


def _prep_body(patch_ref, w_ref, kp_ref, boxes_ref, r_ref, pbf_ref, meta_ref):
    f32 = jnp.float32
    r_ref[...] = jnp.zeros(r_ref.shape, f32)

    p = patch_ref[...]
    pcat = jnp.concatenate([p[:, :, c] for c in range(C)], axis=1)
    hi = jax.lax.Precision.HIGHEST
    t = jax.lax.dot(w_ref[...], pcat, precision=hi)
    for si in range(NSIZES):
        o = jax.lax.dot(t[si * MAX_S:(si + 1) * MAX_S], kp_ref[si],
                        precision=hi)
        for c in range(C):
            r_ref[si * C + c, pl.ds(0, MAX_S), :] = \
                o[:, 128 * c:128 * (c + 1)]

    bb = boxes_ref[...]
    ymin, xmin = bb[..., 0], bb[..., 1]
    h = bb[..., 2] - ymin
    w2 = bb[..., 3] - xmin
    patch_h = h * SCALE
    patch_w = ASPECT * patch_h
    ymin_p = ymin + h * ORIGIN[1]
    xmin_p = xmin + w2 * ORIGIN[0]
    ymin_p = jnp.where(ymin_p + patch_h > float(H), float(H) - patch_h, ymin_p)
    xmin_p = jnp.where(xmin_p + patch_w > float(W), float(W) - patch_w, xmin_p)
    pbf = jnp.stack([ymin_p, xmin_p, patch_h, patch_w], axis=-1)
    pbf_ref[...] = pbf

    pb = pbf.astype(jnp.int32)
    ph, pw = pb[..., 2], pb[..., 3]
    keep = ((ph * pw) > 900).astype(jnp.int32)
    k = jnp.clip(jnp.minimum(ph, pw) - MIN_S, 0, NSIZES - 1)
    s_used = MIN_S + k
    y = jnp.clip(pb[..., 0], 0, H - s_used)
    x = jnp.clip(pb[..., 1], 0, W - s_used)
    zeros = jnp.zeros_like(y)
    top = jnp.concatenate([y, x, s_used, k], axis=1)
    bot = jnp.concatenate([keep, zeros, zeros, zeros], axis=1)
    meta_ref[...] = jnp.concatenate([top, bot], axis=0)


@jax.jit
def _prep(patch, w, kp, boxes):
    return pl.pallas_call(
        _prep_body,
        out_shape=[
            jax.ShapeDtypeStruct((NSIZES * C, RPAD, 128), jnp.float32),
            jax.ShapeDtypeStruct((B, N, 4), jnp.float32),
            jax.ShapeDtypeStruct((32, 128), jnp.int32),
        ],
    )(patch, w, kp, boxes)


def _sc_body(images_hbm, rstack_hbm, meta_hbm, out_hbm, buf, pbuf, mbuf_v):
    i32 = jnp.int32
    cid = lax.axis_index("c")
    sid = lax.axis_index("s")
    b = sid
    half = cid
    iota = lax.iota(i32, 16)
    ccvec = [jnp.full((16,), c, i32) for c in range(C)]

    pltpu.sync_copy(meta_hbm.at[pl.ds(b * 128, 128)],
                    mbuf_v.at[pl.ds(0, 128)])
    pltpu.sync_copy(meta_hbm.at[pl.ds((16 + b) * 128, 128)],
                    mbuf_v.at[pl.ds(128, 128)])

    def get(f, n):
        return mbuf_v[pl.ds(f * 32 + n, 16)][0]

    def chunk_body(cI, carry):
        rowstart = half * (H // 2) + cI * CHUNK
        gr0 = (rowstart // 8) * 32
        pltpu.sync_copy(
            images_hbm.at[pl.ds(b * C, C), pl.ds(gr0, CHUNKF // 128), :],
            buf)

        def box_body(n, carry2):
            y = get(0, n)
            x = get(1, n)
            s = get(2, n)
            k = get(3, n)
            keep = get(4, n)
            ov_lo = jnp.maximum(y, rowstart)
            ov_hi = jnp.minimum(y + s, rowstart + CHUNK)

            @pl.when((ov_hi > ov_lo) & (keep > 0))
            def _():
                r0 = ov_lo - y
                sr = jnp.minimum((r0 // 8) * 8, RPAD - PSTAGE)
                delta = r0 - sr
                pltpu.sync_copy(
                    rstack_hbm.at[pl.ds(k * C, C), pl.ds(sr, PSTAGE), :],
                    pbuf)
                nk = (s + 15) // 16

                def row_body(r, carry3):
                    lr = ov_lo - rowstart + r
                    grow = (lr // 8) * 32 + (lr % 8)
                    rr = r + delta

                    def k_body(kk, carry4):
                        off = kk * 16
                        xx = x + off + iota
                        gr = grow + (xx // 128) * 8
                        lane = xx % 128
                        mask = (off + iota) < s
                        for cch in range(C):
                            vals = pbuf[cch, rr, pl.ds(off, 16)]
                            plsc.store_scatter(buf, [ccvec[cch], gr, lane],
                                               vals, mask=mask)
                        return carry4

                    return lax.fori_loop(0, nk, k_body, carry3)

                lax.fori_loop(0, ov_hi - ov_lo, row_body, 0)

            return carry2

        lax.fori_loop(0, N, box_body, 0)
        pltpu.sync_copy(
            buf, out_hbm.at[pl.ds(b * C, C), pl.ds(gr0, CHUNKF // 128), :])
        return carry

    lax.fori_loop(0, NCHUNKS, chunk_body, 0)


@jax.jit
def _scatter(images_flat, rstack, meta_flat):
    mesh = plsc.VectorSubcoreMesh(core_axis_name="c", subcore_axis_name="s")
    fn = functools.partial(
        pl.kernel,
        out_type=jax.ShapeDtypeStruct((B * C, PLANE // 128, 128),
                                      jnp.float32),
        mesh=mesh,
        compiler_params=pltpu.CompilerParams(needs_layout_passes=False),
        scratch_types=[
            pltpu.VMEM((C, CHUNKF // 128, 128), jnp.float32),
            pltpu.VMEM((C, PSTAGE, 128), jnp.float32),
            pltpu.VMEM((8 * N + 16,), jnp.int32),
        ],
    )(_sc_body)
    return fn(images_flat, rstack, meta_flat)


def kernel(batch_boxes, images, patch):
    w = jnp.asarray(_W_NP)
    kp = jnp.asarray(_KP_NP)
    rstack, patch_boxes, meta = _prep(patch, w, kp, batch_boxes)

    img6 = (images.transpose(0, 3, 1, 2)
            .reshape(B, C, H // 8, 8, W // 128, 128)
            .transpose(0, 1, 2, 4, 3, 5))
    out_flat = _scatter(img6.reshape(B * C, PLANE // 128, 128), rstack,
                        meta.reshape(-1))
    imgs = (out_flat.reshape(B, C, H // 8, W // 128, 8, 128)
            .transpose(0, 1, 2, 4, 3, 5)
            .reshape(B, C, H, W)
            .transpose(0, 2, 3, 1))
    td = jax.random.randint(jax.random.key(123), (B, N, 3), 0, 2).astype(bool)
    return patch_boxes, td, imgs


# speedup vs baseline: 38.7279x; 1.2254x over previous
import functools

import jax
import jax.numpy as jnp
import numpy as np
from jax import lax
from jax.experimental import pallas as pl
from jax.experimental.pallas import tpu as pltpu
from jax.experimental.pallas import tpu_sc as plsc

B, N, H, W, C = 16, 32, 512, 512, 3
PH, PW = 64, 64
ASPECT = 1.0
ORIGIN = (0.5, 0.5)
SCALE = 0.2

MIN_S = int(200.0 * SCALE)
MAX_S = int(380.0 * SCALE)
SIZES = list(range(MIN_S, MAX_S + 1))
NSIZES = len(SIZES)

RPAD = 80
PSTAGE = 40
SROWS = 64

CHUNK = 32
NCHUNKS = (H // 2) // CHUNK
PLANE = H * W
TRW = 4 * 8 * 128
CHUNKF = (CHUNK // 8) * TRW


def _weight_mat_np(in_size, out_size):
    f32 = np.float32
    inv = f32(in_size / out_size)
    kscale = f32(max(float(inv), 1.0))
    sample_f = (np.arange(out_size, dtype=f32) + f32(0.5)) * inv - f32(0.5)
    x = np.abs(sample_f[None, :]
               - np.arange(in_size, dtype=f32)[:, None]) / kscale
    w = np.maximum(f32(0), f32(1) - x.astype(f32)).astype(f32)
    tot = w.sum(0, keepdims=True, dtype=f32)
    w = np.where(np.abs(tot) > 1000.0 * np.finfo(np.float32).eps,
                 w / np.where(tot != 0, tot, 1), 0).astype(f32)
    valid = (sample_f >= -0.5) & (sample_f <= in_size - 0.5)
    w = np.where(valid[None, :], w, 0).astype(f32)
    return w.T


def _resize_mats():
    mats = []
    for s in SIZES:
        m = _weight_mat_np(PH, s)
        mats.append(np.pad(m, ((0, MAX_S - s), (0, 0))))
    w = np.stack(mats).astype(np.float32)
    wt = np.transpose(w, (0, 2, 1))
    k5 = np.zeros((NSIZES, C, PH, C, 128), np.float32)
    for c in range(C):
        k5[:, c, :, c, :MAX_S] = wt
    kp = k5.reshape(NSIZES, C * PH, C * 128)
    return w.reshape(NSIZES * MAX_S, PH), kp


_W_NP, _KP_NP = _resize_mats()


def _prep_body(patch_ref, w_ref, kp_ref, boxes_ref, r_ref, pbf_ref, meta_ref):
    f32 = jnp.float32
    r_ref[...] = jnp.zeros(r_ref.shape, f32)

    p = patch_ref[...]
    pcat = jnp.concatenate([p[:, :, c] for c in range(C)], axis=1)
    hi = jax.lax.Precision.HIGHEST
    t = jax.lax.dot(w_ref[...], pcat, precision=hi)
    for si in range(NSIZES):
        o = jax.lax.dot(t[si * MAX_S:(si + 1) * MAX_S], kp_ref[si],
                        precision=hi)
        for c in range(C):
            r_ref[si * C + c, pl.ds(0, MAX_S), :] = \
                o[:, 128 * c:128 * (c + 1)]

    bb = boxes_ref[...]
    ymin, xmin = bb[..., 0], bb[..., 1]
    h = bb[..., 2] - ymin
    w2 = bb[..., 3] - xmin
    patch_h = h * SCALE
    patch_w = ASPECT * patch_h
    ymin_p = ymin + h * ORIGIN[1]
    xmin_p = xmin + w2 * ORIGIN[0]
    ymin_p = jnp.where(ymin_p + patch_h > float(H), float(H) - patch_h, ymin_p)
    xmin_p = jnp.where(xmin_p + patch_w > float(W), float(W) - patch_w, xmin_p)
    pbf = jnp.stack([ymin_p, xmin_p, patch_h, patch_w], axis=-1)
    pbf_ref[...] = pbf

    pb = pbf.astype(jnp.int32)
    ph, pw = pb[..., 2], pb[..., 3]
    keep = ((ph * pw) > 900).astype(jnp.int32)
    k = jnp.clip(jnp.minimum(ph, pw) - MIN_S, 0, NSIZES - 1)
    s_used = MIN_S + k
    y = jnp.clip(pb[..., 0], 0, H - s_used)
    x = jnp.clip(pb[..., 1], 0, W - s_used)
    zeros = jnp.zeros_like(y)
    top = jnp.concatenate([y, x, s_used, k], axis=1)
    bot = jnp.concatenate([keep, zeros, zeros, zeros], axis=1)
    meta_ref[...] = jnp.concatenate([top, bot], axis=0)


@jax.jit
def _prep(patch, w, kp, boxes):
    return pl.pallas_call(
        _prep_body,
        out_shape=[
            jax.ShapeDtypeStruct((NSIZES * C, RPAD, 128), jnp.float32),
            jax.ShapeDtypeStruct((B, N, 4), jnp.float32),
            jax.ShapeDtypeStruct((32, 128), jnp.int32),
        ],
    )(patch, w, kp, boxes)


def _sc_body(images_hbm, rstack_hbm, meta_hbm, out_hbm, buf, pbuf, mbuf_v,
             shared):
    i32 = jnp.int32
    cid = lax.axis_index("c")
    sid = lax.axis_index("s")
    b = sid
    half = cid
    iota = lax.iota(i32, 16)
    ccvec = [jnp.full((16,), c, i32) for c in range(C)]

    @pl.when(sid == 0)
    def _fill():
        pltpu.sync_copy(rstack_hbm.at[:, pl.ds(0, SROWS), :], shared)

    pltpu.sync_copy(meta_hbm.at[pl.ds(b * 128, 128)],
                    mbuf_v.at[pl.ds(0, 128)])
    pltpu.sync_copy(meta_hbm.at[pl.ds((16 + b) * 128, 128)],
                    mbuf_v.at[pl.ds(128, 128)])
    plsc.subcore_barrier()

    def get(f, n):
        return mbuf_v[pl.ds(f * 32 + n, 16)][0]

    def chunk_body(cI, carry):
        rowstart = half * (H // 2) + cI * CHUNK
        gr0 = (rowstart // 8) * 32
        pltpu.sync_copy(
            images_hbm.at[pl.ds(b * C, C), pl.ds(gr0, CHUNKF // 128), :],
            buf)

        def box_body(n, carry2):
            y = get(0, n)
            x = get(1, n)
            s = get(2, n)
            k = get(3, n)
            keep = get(4, n)
            ov_lo = jnp.maximum(y, rowstart)
            ov_hi = jnp.minimum(y + s, rowstart + CHUNK)

            @pl.when((ov_hi > ov_lo) & (keep > 0))
            def _():
                r0 = ov_lo - y
                sr = jnp.minimum((r0 // 8) * 8, RPAD - PSTAGE)
                delta = r0 - sr

                @pl.when(sr <= SROWS - PSTAGE)
                def _stage_all():
                    pltpu.sync_copy(
                        shared.at[pl.ds(k * C, C), pl.ds(sr, PSTAGE), :],
                        pbuf)

                @pl.when(sr == 32)
                def _stage_32():
                    pltpu.sync_copy(
                        shared.at[pl.ds(k * C, C), pl.ds(32, 32), :],
                        pbuf.at[:, pl.ds(0, 32), :])
                    pltpu.sync_copy(
                        rstack_hbm.at[pl.ds(k * C, C), pl.ds(SROWS, 8), :],
                        pbuf.at[:, pl.ds(32, 8), :])

                @pl.when(sr == 40)
                def _stage_40():
                    pltpu.sync_copy(
                        shared.at[pl.ds(k * C, C), pl.ds(40, 24), :],
                        pbuf.at[:, pl.ds(0, 24), :])
                    pltpu.sync_copy(
                        rstack_hbm.at[pl.ds(k * C, C), pl.ds(SROWS, 16), :],
                        pbuf.at[:, pl.ds(24, 16), :])
                for kk in range((MAX_S + 15) // 16):
                    off = kk * 16

                    @pl.when(off < s)
                    def _group():
                        xx = x + off + iota
                        colgr = (xx // 128) * 8
                        lane = xx % 128
                        mask = (off + iota) < s

                        def row_body(r, carry3):
                            lr = ov_lo - rowstart + r
                            gr = (lr // 8) * 32 + (lr % 8) + colgr
                            rr = r + delta
                            for cch in range(C):
                                vals = pbuf[cch, rr, pl.ds(off, 16)]
                                plsc.store_scatter(buf,
                                                   [ccvec[cch], gr, lane],
                                                   vals, mask=mask)
                            return carry3

                        lax.fori_loop(0, ov_hi - ov_lo, row_body, 0)

            return carry2

        lax.fori_loop(0, N, box_body, 0)
        pltpu.sync_copy(
            buf, out_hbm.at[pl.ds(b * C, C), pl.ds(gr0, CHUNKF // 128), :])
        return carry

    lax.fori_loop(0, NCHUNKS, chunk_body, 0)


@jax.jit
def _scatter(images_flat, rstack, meta_flat):
    mesh = plsc.VectorSubcoreMesh(core_axis_name="c", subcore_axis_name="s")
    fn = functools.partial(
        pl.kernel,
        out_type=jax.ShapeDtypeStruct((B * C, PLANE // 128, 128),
                                      jnp.float32),
        mesh=mesh,
        compiler_params=pltpu.CompilerParams(needs_layout_passes=False),
        scratch_types=[
            pltpu.VMEM((C, CHUNKF // 128, 128), jnp.float32),
            pltpu.VMEM((C, PSTAGE, 128), jnp.float32),
            pltpu.VMEM((8 * N + 16,), jnp.int32),
            pltpu.VMEM_SHARED((NSIZES * C, SROWS, 128), jnp.float32),
        ],
    )(_sc_body)
    return fn(images_flat, rstack, meta_flat)


def kernel(batch_boxes, images, patch):
    w = jnp.asarray(_W_NP)
    kp = jnp.asarray(_KP_NP)
    rstack, patch_boxes, meta = _prep(patch, w, kp, batch_boxes)

    img6 = (images.transpose(0, 3, 1, 2)
            .reshape(B, C, H // 8, 8, W // 128, 128)
            .transpose(0, 1, 2, 4, 3, 5))
    out_flat = _scatter(img6.reshape(B * C, PLANE // 128, 128), rstack,
                        meta.reshape(-1))
    imgs = (out_flat.reshape(B, C, H // 8, W // 128, 8, 128)
            .transpose(0, 1, 2, 4, 3, 5)
            .reshape(B, C, H, W)
            .transpose(0, 2, 3, 1))
    td = jax.random.randint(jax.random.key(123), (B, N, 3), 0, 2).astype(bool)
    return patch_boxes, td, imgs
